# Initial kernel scaffold; baseline (speedup 1.0000x reference)
import jax
import jax.numpy as jnp
from jax.experimental import pallas as pl


def kernel(x, edge_index, W, b):
    raise NotImplementedError("write your pallas kernel here")


# speedup vs baseline: 45.7299x; 45.7299x over previous
import functools

import jax
import jax.numpy as jnp
from jax import lax
from jax.experimental import pallas as pl
from jax.experimental.pallas import tpu as pltpu
from jax.experimental.pallas import tpu_sc as plsc

N_NODES = 10000
D = 128
N_LAYERS = 8
N_PAD = 10240
DUMMY = N_NODES

NC, NS = 2, 16
NW = NC * NS
CHUNK = 128
NCHUNK = 81
PER_TILE = CHUNK * NCHUNK
E_PAD = PER_TILE * NW
ROWS_PER_TILE = N_PAD // NS

_mesh = plsc.VectorSubcoreMesh(core_axis_name="c", subcore_axis_name="s")


@functools.partial(
    pl.kernel,
    mesh=_mesh,
    out_type=jax.ShapeDtypeStruct((NC, N_PAD), jnp.float32),
    scratch_types=[
        pltpu.VMEM((CHUNK,), jnp.int32),
        pltpu.VMEM((CHUNK,), jnp.float32),
        pltpu.VMEM((ROWS_PER_TILE,), jnp.float32),
        pltpu.VMEM_SHARED((N_PAD,), jnp.float32),
    ],
)
def _deg_kernel(dst_hbm, out_hbm, idx_v, ones_v, node_v, acc):
    c = lax.axis_index("c")
    s = lax.axis_index("s")
    for i in range(ROWS_PER_TILE // 16):
        node_v[pl.ds(16 * i, 16)] = jnp.zeros((16,), jnp.float32)
    for i in range(CHUNK // 16):
        ones_v[pl.ds(16 * i, 16)] = jnp.ones((16,), jnp.float32)

    nbase = s * ROWS_PER_TILE
    pltpu.sync_copy(node_v, acc.at[pl.ds(nbase, ROWS_PER_TILE)])
    plsc.subcore_barrier()
    ebase = (c * NS + s) * PER_TILE

    def chunk_body(j, carry):
        pltpu.sync_copy(dst_hbm.at[pl.ds(ebase + j * CHUNK, CHUNK)], idx_v)
        pltpu.sync_copy(ones_v, acc.at[idx_v], add=True)
        return carry

    lax.fori_loop(0, NCHUNK, chunk_body, 0)
    plsc.subcore_barrier()
    pltpu.sync_copy(acc.at[pl.ds(nbase, ROWS_PER_TILE)], node_v)
    pltpu.sync_copy(node_v, out_hbm.at[c, pl.ds(nbase, ROWS_PER_TILE)])


@functools.partial(
    pl.kernel,
    mesh=_mesh,
    out_type=jax.ShapeDtypeStruct((NC, N_PAD, D), jnp.float32),
    scratch_types=[
        pltpu.VMEM((CHUNK,), jnp.int32),
        pltpu.VMEM((CHUNK,), jnp.int32),
        pltpu.VMEM((CHUNK, D), jnp.float32),
        pltpu.VMEM_SHARED((N_PAD, D), jnp.float32),
        pltpu.SemaphoreType.DMA,
    ],
)
def _agg_kernel(y_hbm, src_hbm, dst_hbm, out_hbm, sidx_v, didx_v, rows_v, acc,
                sem):
    c = lax.axis_index("c")
    s = lax.axis_index("s")

    def zero_row(i, carry):
        for k in range(D // 16):
            rows_v[i, pl.ds(16 * k, 16)] = jnp.zeros((16,), jnp.float32)
        return carry

    lax.fori_loop(0, CHUNK, zero_row, 0)

    nbase = s * ROWS_PER_TILE
    for k in range(ROWS_PER_TILE // CHUNK):
        pltpu.sync_copy(rows_v, acc.at[pl.ds(nbase + k * CHUNK, CHUNK)])
    plsc.subcore_barrier()
    ebase = (c * NS + s) * PER_TILE

    def chunk_body(j, carry):
        eb = ebase + j * CHUNK
        pltpu.sync_copy(src_hbm.at[pl.ds(eb, CHUNK)], sidx_v)
        pltpu.sync_copy(dst_hbm.at[pl.ds(eb, CHUNK)], didx_v)
        pltpu.async_copy(y_hbm.at[sidx_v], rows_v, sem).wait()
        pltpu.sync_copy(rows_v, acc.at[didx_v], add=True)
        return carry

    lax.fori_loop(0, NCHUNK, chunk_body, 0)
    plsc.subcore_barrier()
    for k in range(ROWS_PER_TILE // CHUNK):
        pltpu.sync_copy(acc.at[pl.ds(nbase + k * CHUNK, CHUNK)], rows_v)
        pltpu.sync_copy(rows_v, out_hbm.at[c, pl.ds(nbase + k * CHUNK, CHUNK)])


def _dis(deg):
    return jnp.where(deg > 0, lax.rsqrt(jnp.maximum(deg, 1e-12)), 0.0)


def _scale_body(x_ref, d0_ref, d1_ref, y_ref):
    deg = d0_ref[...] + d1_ref[...]
    y_ref[...] = x_ref[...] * _dis(deg)


_ROW_BLK = 512
_N_BLKS = N_PAD // _ROW_BLK


def _scale_call(xp, d0, d1):
    return pl.pallas_call(
        _scale_body,
        grid=(_N_BLKS,),
        in_specs=[
            pl.BlockSpec((_ROW_BLK, D), lambda i: (i, 0)),
            pl.BlockSpec((_ROW_BLK, 1), lambda i: (i, 0)),
            pl.BlockSpec((_ROW_BLK, 1), lambda i: (i, 0)),
        ],
        out_specs=pl.BlockSpec((_ROW_BLK, D), lambda i: (i, 0)),
        out_shape=jax.ShapeDtypeStruct((N_PAD, D), jnp.float32),
    )(xp, d0, d1)


def _mm_body(z0_ref, z1_ref, d0_ref, d1_ref, w_ref, b_ref, o_ref):
    deg = d0_ref[...] + d1_ref[...]
    xa = (z0_ref[...] + z1_ref[...]) * _dis(deg)
    acc = lax.dot_general(
        xa, w_ref[...], (((1,), (0,)), ((), ())),
        preferred_element_type=jnp.float32,
    )
    o_ref[...] = jax.nn.sigmoid(acc + b_ref[...])


def _mm_call(z0, z1, d0, d1, w_cat, b_cat):
    return pl.pallas_call(
        _mm_body,
        grid=(_N_BLKS,),
        in_specs=[
            pl.BlockSpec((_ROW_BLK, D), lambda i: (i, 0)),
            pl.BlockSpec((_ROW_BLK, D), lambda i: (i, 0)),
            pl.BlockSpec((_ROW_BLK, 1), lambda i: (i, 0)),
            pl.BlockSpec((_ROW_BLK, 1), lambda i: (i, 0)),
            pl.BlockSpec((D, N_LAYERS * D), lambda i: (0, 0)),
            pl.BlockSpec((1, N_LAYERS * D), lambda i: (0, 0)),
        ],
        out_specs=pl.BlockSpec((_ROW_BLK, N_LAYERS * D), lambda i: (i, 0)),
        out_shape=jax.ShapeDtypeStruct((N_PAD, N_LAYERS * D), jnp.float32),
    )(z0, z1, d0, d1, w_cat, b_cat)


@jax.jit
def kernel(x, edge_index, W, b):
    loop = jnp.arange(N_NODES, dtype=edge_index.dtype)
    n_extra = E_PAD - edge_index.shape[1] - N_NODES
    src = jnp.concatenate(
        [edge_index[0], loop, jnp.zeros((n_extra,), edge_index.dtype)])
    dst = jnp.concatenate(
        [edge_index[1], loop, jnp.full((n_extra,), DUMMY, edge_index.dtype)])

    degp = _deg_kernel(dst)
    d0 = degp[0].reshape(N_PAD, 1)
    d1 = degp[1].reshape(N_PAD, 1)

    xp = jnp.pad(x, ((0, N_PAD - N_NODES), (0, 0)))
    y = _scale_call(xp, d0, d1)

    z = _agg_kernel(y, src, dst)

    w_cat = jnp.transpose(W, (1, 0, 2)).reshape(D, N_LAYERS * D)
    b_cat = b.reshape(1, N_LAYERS * D)
    out = _mm_call(z[0], z[1], d0, d1, w_cat, b_cat)
    return out[:N_NODES]
